# initial kernel scaffold (unmeasured)
import jax
import jax.numpy as jnp
from jax import lax
from jax.experimental import pallas as pl
from jax.experimental.pallas import tpu as pltpu

N_DEV = 32
M, K_SHARD, N = 4096, 128, 2048
CHUNK = M // N_DEV
N_STEPS = 2 * (N_DEV - 1)


def kernel(x, w_mat, scale_x, scale_w):
    def body(x_ref, w_ref, sx_ref, sw_ref, out_ref,
             rs_buf, send_sems, recv_sems, credit_sem):
        my = lax.axis_index("i")
        left = lax.rem(my + N_DEV - 1, N_DEV)
        right = lax.rem(my + 1, N_DEV)

        barrier_sem = pltpu.get_barrier_semaphore()
        for nbr in (left, right):
            pl.semaphore_signal(barrier_sem, inc=1, device_id=(nbr,),
                                device_id_type=pl.DeviceIdType.MESH)
        pl.semaphore_wait(barrier_sem, 2)

        acc = lax.dot_general(
            x_ref[...], w_ref[...], (((1,), (0,)), ((), ())),
            preferred_element_type=jnp.int32)
        out_ref[...] = acc.astype(jnp.float32)

        def chunk_slice(c):
            return (pl.ds(c * CHUNK, CHUNK), slice(None))

        def step_rdma(h, src_sl, dst_is_out, dst_sl_or_slot):
            send_slot = h % 2
            recv_slot = (h + 1) % 2
            if dst_is_out:
                dst = out_ref.at[dst_sl_or_slot]
            else:
                dst = rs_buf.at[dst_sl_or_slot]
            rdma = pltpu.make_async_remote_copy(
                src_ref=out_ref.at[src_sl],
                dst_ref=dst,
                send_sem=send_sems.at[send_slot],
                recv_sem=recv_sems.at[recv_slot],
                device_id=(right,),
                device_id_type=pl.DeviceIdType.MESH,
            )
            if h >= 2:
                pl.semaphore_wait(credit_sem, 1)
            rdma.start()
            rdma.wait()

        def send_credit(h):
            if h < N_STEPS - 2:
                pl.semaphore_signal(credit_sem, inc=1, device_id=(left,),
                                    device_id_type=pl.DeviceIdType.MESH)

        for h in range(N_DEV - 1):
            c_send = lax.rem(my + (2 * N_DEV - h), N_DEV)
            c_recv = lax.rem(my + (2 * N_DEV - h - 1), N_DEV)
            recv_slot = (h + 1) % 2
            step_rdma(h, chunk_slice(c_send), False, recv_slot)
            sl = chunk_slice(c_recv)
            out_ref[sl] = out_ref[sl] + rs_buf[recv_slot]
            send_credit(h)

        own = lax.rem(my + 1, N_DEV)
        sl = chunk_slice(own)
        s = sx_ref[0] * sw_ref[0]
        y = out_ref[sl] * s
        out_ref[sl] = y / (1.0 + jnp.exp(-y))

        for g in range(N_DEV - 1):
            h = (N_DEV - 1) + g
            c = lax.rem(my + (2 * N_DEV + 1 - g), N_DEV)
            step_rdma(h, chunk_slice(c), True, chunk_slice(c))
            send_credit(h)

    grid_spec = pltpu.PrefetchScalarGridSpec(
        num_scalar_prefetch=0,
        in_specs=[
            pl.BlockSpec(memory_space=pltpu.VMEM),
            pl.BlockSpec(memory_space=pltpu.VMEM),
            pl.BlockSpec(memory_space=pltpu.SMEM),
            pl.BlockSpec(memory_space=pltpu.SMEM),
        ],
        out_specs=pl.BlockSpec(memory_space=pltpu.VMEM),
        scratch_shapes=[
            pltpu.VMEM((2, CHUNK, N), jnp.float32),
            pltpu.SemaphoreType.DMA((2,)),
            pltpu.SemaphoreType.DMA((2,)),
            pltpu.SemaphoreType.REGULAR,
        ],
    )
    return pl.pallas_call(
        body,
        grid_spec=grid_spec,
        out_shape=jax.ShapeDtypeStruct((M, N), jnp.float32),
        compiler_params=pltpu.CompilerParams(
            collective_id=0,
            vmem_limit_bytes=100 * 1024 * 1024,
        ),
    )(x, w_mat, scale_x, scale_w)


# baseline (device time: 859235 ns/iter reference)
import jax
import jax.numpy as jnp
from jax import lax
from jax.experimental import pallas as pl
from jax.experimental.pallas import tpu as pltpu

N_DEV = 32
M, K_SHARD, N = 4096, 128, 2048
CHUNK = M // N_DEV
N_STEPS = 2 * (N_DEV - 1)


def kernel(x, w_mat, scale_x, scale_w):
    def body(x_ref, w_ref, sx_ref, sw_ref, out_ref,
             rs_buf, send_sems, recv_sems, credit_sem):
        my = lax.axis_index("i")
        left = lax.rem(my + N_DEV - 1, N_DEV)
        right = lax.rem(my + 1, N_DEV)

        barrier_sem = pltpu.get_barrier_semaphore()
        for nbr in (left, right):
            pl.semaphore_signal(barrier_sem, inc=1, device_id=(nbr,),
                                device_id_type=pl.DeviceIdType.MESH)
        pl.semaphore_wait(barrier_sem, 2)

        acc = lax.dot_general(
            x_ref[...], w_ref[...], (((1,), (0,)), ((), ())),
            preferred_element_type=jnp.int32)
        out_ref[...] = acc.astype(jnp.float32)

        def chunk_slice(c):
            return (pl.ds(c * CHUNK, CHUNK), slice(None))

        def step_rdma(h, src_sl, dst_is_out, dst_sl_or_slot):
            send_slot = h % 2
            recv_slot = (h + 1) % 2
            if dst_is_out:
                dst = out_ref.at[dst_sl_or_slot]
            else:
                dst = rs_buf.at[dst_sl_or_slot]
            rdma = pltpu.make_async_remote_copy(
                src_ref=out_ref.at[src_sl],
                dst_ref=dst,
                send_sem=send_sems.at[send_slot],
                recv_sem=recv_sems.at[recv_slot],
                device_id=(right,),
                device_id_type=pl.DeviceIdType.MESH,
            )
            if h >= 2:
                pl.semaphore_wait(credit_sem, 1)
            rdma.start()
            rdma.wait()

        def send_credit(h):
            if h < N_STEPS - 2:
                pl.semaphore_signal(credit_sem, inc=1, device_id=(left,),
                                    device_id_type=pl.DeviceIdType.MESH)

        for h in range(N_DEV - 1):
            c_send = lax.rem(my + (2 * N_DEV - h), N_DEV)
            c_recv = lax.rem(my + (2 * N_DEV - h - 1), N_DEV)
            recv_slot = (h + 1) % 2
            step_rdma(h, chunk_slice(c_send), False, recv_slot)
            sl = chunk_slice(c_recv)
            out_ref[sl] = out_ref[sl] + rs_buf[recv_slot]
            send_credit(h)

        own = lax.rem(my + 1, N_DEV)
        sl = chunk_slice(own)
        s = sx_ref[0] * sw_ref[0]
        y = out_ref[sl] * s
        out_ref[sl] = y / (1.0 + jnp.exp(-y))

        for g in range(N_DEV - 1):
            h = (N_DEV - 1) + g
            c = lax.rem(my + (2 * N_DEV + 1 - g), N_DEV)
            step_rdma(h, chunk_slice(c), True, chunk_slice(c))
            send_credit(h)

    return pl.pallas_call(
        body,
        out_shape=jax.ShapeDtypeStruct((M, N), jnp.float32),
        in_specs=[
            pl.BlockSpec(memory_space=pltpu.VMEM),
            pl.BlockSpec(memory_space=pltpu.VMEM),
            pl.BlockSpec(memory_space=pltpu.SMEM),
            pl.BlockSpec(memory_space=pltpu.SMEM),
        ],
        out_specs=pl.BlockSpec(memory_space=pltpu.VMEM),
        scratch_shapes=[
            pltpu.VMEM((2, CHUNK, N), jnp.float32),
            pltpu.SemaphoreType.DMA((2,)),
            pltpu.SemaphoreType.DMA((2,)),
            pltpu.SemaphoreType.REGULAR,
        ],
        compiler_params=pltpu.CompilerParams(
            collective_id=0,
            vmem_limit_bytes=100 * 1024 * 1024,
        ),
    )(x, w_mat, scale_x, scale_w)


# device time: 507078 ns/iter; 1.6945x vs baseline; 1.6945x over previous
import numpy as np

import jax
import jax.numpy as jnp
from jax import lax
from jax.experimental import pallas as pl
from jax.experimental.pallas import tpu as pltpu

N_DEV = 32
M, K_SHARD, N = 4096, 128, 2048
CHUNK = M // N_DEV
HALF = N // 2
N_STEPS = 2 * (N_DEV - 1)

_P = [(0, 0), (1, 0), (2, 0), (3, 0),
      (3, 1), (2, 1), (1, 1), (0, 1),
      (0, 2), (1, 2), (2, 2), (3, 2),
      (3, 3), (2, 3), (1, 3), (0, 3)]
_RING_COORDS = [(0, y, z) for (y, z) in _P] + [(1, y, z) for (y, z) in reversed(_P)]

_PLANE_IDX = {(0, 0): 0, (1, 0): 1, (1, 1): 2, (0, 1): 3,
              (0, 2): 4, (1, 2): 5, (1, 3): 6, (0, 3): 7}
_RING = [8 * z + _PLANE_IDX[(x, y)] for (x, y, z) in _RING_COORDS]
_POS = [0] * N_DEV
for _r, _l in enumerate(_RING):
    _POS[_l] = _r
assert sorted(_RING) == list(range(N_DEV))


def kernel(x, w_mat, scale_x, scale_w):
    my = lax.axis_index("i")
    pos = jnp.asarray(_POS, jnp.int32)[my]
    ring = jnp.asarray(_RING, jnp.int32)
    right = ring[lax.rem(pos + 1, N_DEV)]
    left = ring[lax.rem(pos + N_DEV - 1, N_DEV)]
    pos_a = pos.reshape(1)
    left_a = left.reshape(1)
    right_a = right.reshape(1)

    def body(x_ref, w_ref, sx_ref, sw_ref, pos_ref, left_ref, right_ref,
             out_ref, cw_buf, ccw_buf,
             send_cw, recv_cw, send_ccw, recv_ccw, credit_cw, credit_ccw):
        pos = pos_ref[0]
        left = left_ref[0]
        right = right_ref[0]

        barrier_sem = pltpu.get_barrier_semaphore()
        for nbr in (left, right):
            pl.semaphore_signal(barrier_sem, inc=1, device_id=(nbr,),
                                device_id_type=pl.DeviceIdType.MESH)
        pl.semaphore_wait(barrier_sem, 2)

        acc = lax.dot_general(
            x_ref[...], w_ref[...], (((1,), (0,)), ((), ())),
            preferred_element_type=jnp.int32)
        out_ref[...] = acc.astype(jnp.float32)

        def cw_slice(c):
            return (pl.ds(c * CHUNK, CHUNK), pl.ds(0, HALF))

        def ccw_slice(c):
            return (pl.ds(c * CHUNK, CHUNK), pl.ds(HALF, HALF))

        def make_pair(h, cw_src, cw_dst, ccw_src, ccw_dst):
            ss, rs = h % 2, (h + 1) % 2
            rdma_cw = pltpu.make_async_remote_copy(
                src_ref=out_ref.at[cw_src], dst_ref=cw_dst,
                send_sem=send_cw.at[ss], recv_sem=recv_cw.at[rs],
                device_id=(right,), device_id_type=pl.DeviceIdType.MESH)
            rdma_ccw = pltpu.make_async_remote_copy(
                src_ref=out_ref.at[ccw_src], dst_ref=ccw_dst,
                send_sem=send_ccw.at[ss], recv_sem=recv_ccw.at[rs],
                device_id=(left,), device_id_type=pl.DeviceIdType.MESH)
            if h >= 2:
                pl.semaphore_wait(credit_cw, 1)
                pl.semaphore_wait(credit_ccw, 1)
            rdma_cw.start()
            rdma_ccw.start()
            return rdma_cw, rdma_ccw

        def send_credits(h):
            if h < N_STEPS - 2:
                pl.semaphore_signal(credit_cw, inc=1, device_id=(left,),
                                    device_id_type=pl.DeviceIdType.MESH)
                pl.semaphore_signal(credit_ccw, inc=1, device_id=(right,),
                                    device_id_type=pl.DeviceIdType.MESH)

        for h in range(N_DEV - 1):
            rs = (h + 1) % 2
            cw_send = lax.rem(pos + (2 * N_DEV - h), N_DEV)
            cw_recv = lax.rem(pos + (2 * N_DEV - h - 1), N_DEV)
            ccw_send = lax.rem(pos + h, N_DEV)
            ccw_recv = lax.rem(pos + h + 1, N_DEV)
            rdma_cw, rdma_ccw = make_pair(
                h, cw_slice(cw_send), cw_buf.at[rs],
                ccw_slice(ccw_send), ccw_buf.at[rs])
            rdma_cw.wait()
            sl = cw_slice(cw_recv)
            out_ref[sl] = out_ref[sl] + cw_buf[rs]
            rdma_ccw.wait()
            sl = ccw_slice(ccw_recv)
            out_ref[sl] = out_ref[sl] + ccw_buf[rs]
            send_credits(h)

        s = sx_ref[0] * sw_ref[0]
        for sl in (cw_slice(lax.rem(pos + 1, N_DEV)),
                   ccw_slice(lax.rem(pos + N_DEV - 1, N_DEV))):
            y = out_ref[sl] * s
            out_ref[sl] = y / (1.0 + jnp.exp(-y))

        for g in range(N_DEV - 1):
            h = (N_DEV - 1) + g
            cw_c = lax.rem(pos + (2 * N_DEV + 1 - g), N_DEV)
            ccw_c = lax.rem(pos + (N_DEV - 1 + g), N_DEV)
            rdma_cw, rdma_ccw = make_pair(
                h, cw_slice(cw_c), out_ref.at[cw_slice(cw_c)],
                ccw_slice(ccw_c), out_ref.at[ccw_slice(ccw_c)])
            rdma_cw.wait()
            rdma_ccw.wait()
            send_credits(h)

    return pl.pallas_call(
        body,
        out_shape=jax.ShapeDtypeStruct((M, N), jnp.float32),
        in_specs=[
            pl.BlockSpec(memory_space=pltpu.VMEM),
            pl.BlockSpec(memory_space=pltpu.VMEM),
            pl.BlockSpec(memory_space=pltpu.SMEM),
            pl.BlockSpec(memory_space=pltpu.SMEM),
            pl.BlockSpec(memory_space=pltpu.SMEM),
            pl.BlockSpec(memory_space=pltpu.SMEM),
            pl.BlockSpec(memory_space=pltpu.SMEM),
        ],
        out_specs=pl.BlockSpec(memory_space=pltpu.VMEM),
        scratch_shapes=[
            pltpu.VMEM((2, CHUNK, HALF), jnp.float32),
            pltpu.VMEM((2, CHUNK, HALF), jnp.float32),
            pltpu.SemaphoreType.DMA((2,)),
            pltpu.SemaphoreType.DMA((2,)),
            pltpu.SemaphoreType.DMA((2,)),
            pltpu.SemaphoreType.DMA((2,)),
            pltpu.SemaphoreType.REGULAR,
            pltpu.SemaphoreType.REGULAR,
        ],
        compiler_params=pltpu.CompilerParams(
            collective_id=0,
            vmem_limit_bytes=100 * 1024 * 1024,
        ),
    )(x, w_mat, scale_x, scale_w, pos_a, left_a, right_a)


# device time: 400749 ns/iter; 2.1441x vs baseline; 1.2653x over previous
import jax
import jax.numpy as jnp
from jax import lax
from jax.experimental import pallas as pl
from jax.experimental.pallas import tpu as pltpu

N_DEV = 32
M, K_SHARD, N = 4096, 128, 2048
CHUNK = M // N_DEV
N_RINGS = 4
QW = N // N_RINGS
N_STEPS = 2 * (N_DEV - 1)

_P = [(0, 0), (1, 0), (2, 0), (3, 0),
      (3, 1), (2, 1), (1, 1), (0, 1),
      (0, 2), (1, 2), (2, 2), (3, 2),
      (3, 3), (2, 3), (1, 3), (0, 3)]
_RING_COORDS = [(0, y, z) for (y, z) in _P] + [(1, y, z) for (y, z) in reversed(_P)]

_PLANE_IDX = {(0, 0): 0, (1, 0): 1, (1, 1): 2, (0, 1): 3,
              (0, 2): 4, (1, 2): 5, (1, 3): 6, (0, 3): 7}
_RING = [8 * z + _PLANE_IDX[(x, y)] for (x, y, z) in _RING_COORDS]
_POS = [0] * N_DEV
for _r, _l in enumerate(_RING):
    _POS[_l] = _r
assert sorted(_RING) == list(range(N_DEV))


def kernel(x, w_mat, scale_x, scale_w):
    my = lax.axis_index("i")
    pos = jnp.asarray(_POS, jnp.int32)[my]
    ring = jnp.asarray(_RING, jnp.int32)
    right = ring[lax.rem(pos + 1, N_DEV)]
    left = ring[lax.rem(pos + N_DEV - 1, N_DEV)]
    pos_a = pos.reshape(1)
    left_a = left.reshape(1)
    right_a = right.reshape(1)

    def body(x_ref, w_ref, sx_ref, sw_ref, pos_ref, left_ref, right_ref,
             out_ref, buf0, buf1, buf2, buf3, send_sems, recv_sems,
             credit0, credit1, credit2, credit3):
        pos = pos_ref[0]
        left = left_ref[0]
        right = right_ref[0]

        barrier_sem = pltpu.get_barrier_semaphore()
        for nbr in (left, right):
            pl.semaphore_signal(barrier_sem, inc=1, device_id=(nbr,),
                                device_id_type=pl.DeviceIdType.MESH)
        pl.semaphore_wait(barrier_sem, 2)

        acc = lax.dot_general(
            x_ref[...], w_ref[...], (((1,), (0,)), ((), ())),
            preferred_element_type=jnp.int32)
        out_ref[...] = acc.astype(jnp.float32)

        bufs = (buf0, buf1, buf2, buf3)
        credits = (credit0, credit1, credit2, credit3)
        cw = (True, True, False, False)
        s = sx_ref[0] * sw_ref[0]

        def qslice(k, c):
            return (pl.ds(c * CHUNK, CHUNK), pl.ds(k * QW, QW))

        def chunk_of(k, h):
            if h < N_DEV - 1:
                d = (2 * N_DEV - h) if cw[k] else h
            else:
                g = h - (N_DEV - 1)
                d = (2 * N_DEV + 1 - g) if cw[k] else (N_DEV - 1 + g)
            return lax.rem(pos + d, N_DEV)

        def make_rdma(k, h):
            c = chunk_of(k, h)
            if h < N_DEV - 1:
                dst = bufs[k].at[(h + 1) % 2]
            else:
                dst = out_ref.at[qslice(k, c)]
            return pltpu.make_async_remote_copy(
                src_ref=out_ref.at[qslice(k, c)],
                dst_ref=dst,
                send_sem=send_sems.at[k, h % 2],
                recv_sem=recv_sems.at[k, (h + 1) % 2],
                device_id=(right if cw[k] else left,),
                device_id_type=pl.DeviceIdType.MESH)

        def consume(k, h):
            if h < N_DEV - 1:
                c = lax.rem(pos + ((2 * N_DEV - h - 1) if cw[k] else (h + 1)),
                            N_DEV)
                sl = qslice(k, c)
                out_ref[sl] = out_ref[sl] + bufs[k][(h + 1) % 2]
                if h == N_DEV - 2:
                    own = lax.rem(pos + (1 if cw[k] else N_DEV - 1), N_DEV)
                    osl = qslice(k, own)
                    y = out_ref[osl] * s
                    out_ref[osl] = y / (1.0 + jnp.exp(-y))

        def send_credit(k, h):
            if h < N_STEPS - 2:
                pl.semaphore_signal(credits[k], inc=1,
                                    device_id=(left if cw[k] else right,),
                                    device_id_type=pl.DeviceIdType.MESH)

        inflight = [make_rdma(k, 0) for k in range(N_RINGS)]
        for k in range(N_RINGS):
            inflight[k].start()
        for h in range(N_STEPS):
            for k in range(N_RINGS):
                inflight[k].wait()
                consume(k, h)
                send_credit(k, h)
                if h + 1 < N_STEPS:
                    if h + 1 >= 2:
                        pl.semaphore_wait(credits[k], 1)
                    nxt = make_rdma(k, h + 1)
                    nxt.start()
                    inflight[k] = nxt

    return pl.pallas_call(
        body,
        out_shape=jax.ShapeDtypeStruct((M, N), jnp.float32),
        in_specs=[
            pl.BlockSpec(memory_space=pltpu.VMEM),
            pl.BlockSpec(memory_space=pltpu.VMEM),
            pl.BlockSpec(memory_space=pltpu.SMEM),
            pl.BlockSpec(memory_space=pltpu.SMEM),
            pl.BlockSpec(memory_space=pltpu.SMEM),
            pl.BlockSpec(memory_space=pltpu.SMEM),
            pl.BlockSpec(memory_space=pltpu.SMEM),
        ],
        out_specs=pl.BlockSpec(memory_space=pltpu.VMEM),
        scratch_shapes=[
            pltpu.VMEM((2, CHUNK, QW), jnp.float32),
            pltpu.VMEM((2, CHUNK, QW), jnp.float32),
            pltpu.VMEM((2, CHUNK, QW), jnp.float32),
            pltpu.VMEM((2, CHUNK, QW), jnp.float32),
            pltpu.SemaphoreType.DMA((N_RINGS, 2)),
            pltpu.SemaphoreType.DMA((N_RINGS, 2)),
            pltpu.SemaphoreType.REGULAR,
            pltpu.SemaphoreType.REGULAR,
            pltpu.SemaphoreType.REGULAR,
            pltpu.SemaphoreType.REGULAR,
        ],
        compiler_params=pltpu.CompilerParams(
            collective_id=0,
            vmem_limit_bytes=100 * 1024 * 1024,
        ),
    )(x, w_mat, scale_x, scale_w, pos_a, left_a, right_a)
